# baseline (speedup 1.0000x reference)
import functools

import jax
import jax.numpy as jnp
from jax import lax
from jax.experimental import pallas as pl
from jax.experimental.pallas import tpu as pltpu
from jax.experimental.pallas import tpu_sc as plsc

_K = 16
_TR = 256
_TC = 256
_W = _K + _TC
_BIG = 2**30
_INF = jnp.inf
_NW = 32


def _knn_body(lo_ref, cnt_ref, coords_ref, coordsT_ref, brow_ref, bcol_ref,
              out_idx_ref):
    i = pl.program_id(0)
    dstT = coordsT_ref[:, pl.ds(i * _TR, _TR)]
    bd = bcol_ref[:, pl.ds(i * _TR, _TR)]
    sqd = jnp.sum(dstT * dstT, axis=0, keepdims=True)

    sub16 = jax.lax.broadcasted_iota(jnp.int32, (_K, _TR), 0)
    subT = jax.lax.broadcasted_iota(jnp.int32, (_TC, _TR), 0)

    def col_body(j, carry):
        bval, bidx = carry
        c0 = (lo_ref[i] + j) * _TC
        src = coords_ref[pl.ds(c0, _TC), :]
        bs = brow_ref[pl.ds(c0, _TC), :]
        m2 = jnp.dot(src, dstT, preferred_element_type=jnp.float32)
        sqs = jnp.sum(src * src, axis=1, keepdims=True)
        d2 = (sqd + sqs) - 2.0 * m2
        dv = jnp.where(bs == bd, d2, _INF)
        k2 = jnp.zeros((1, _TR), jnp.int32)
        vals, idxs = [], []
        for r in range(_K):
            m1 = jnp.min(dv, axis=0, keepdims=True)
            sel2 = sub16 == k2
            m2r = jnp.min(jnp.where(sel2, bval, _INF),
                          axis=0, keepdims=True)
            gi = jnp.min(jnp.where(sel2, bidx, _BIG),
                         axis=0, keepdims=True)
            use_bv = m2r <= m1
            t = jnp.where(use_bv, jnp.nan, m1)
            eqc = dv == t
            pos1 = jnp.min(jnp.where(eqc, subT, _BIG),
                           axis=0, keepdims=True)
            vals.append(jnp.where(use_bv, m2r, m1))
            idxs.append(jnp.where(use_bv, gi, c0 + pos1))
            k2 = k2 + use_bv.astype(jnp.int32)
            if r < _K - 1:
                dv = jnp.where(eqc, _INF, dv)
        return (jnp.concatenate(vals, axis=0), jnp.concatenate(idxs, axis=0))

    init = (jnp.full((_K, _TR), _INF, jnp.float32),
            jnp.full((_K, _TR), _BIG, jnp.int32))
    _, bi = jax.lax.fori_loop(0, cnt_ref[i], col_body, init)
    out_idx_ref[...] = bi


def _make_sc_compact(npad):
    npn = npad // _NW
    ngr = npn // _K
    mesh = plsc.VectorSubcoreMesh(core_axis_name="c", subcore_axis_name="s")

    @functools.partial(
        pl.kernel, mesh=mesh,
        compiler_params=pltpu.CompilerParams(needs_layout_passes=False),
        out_type=[
            jax.ShapeDtypeStruct((npad * (_K - 1),), jnp.int32),
            jax.ShapeDtypeStruct((npad * (_K - 1),), jnp.int32),
            jax.ShapeDtypeStruct((npad,), jnp.int32),
        ],
        scratch_types=[
            pltpu.VMEM((npn * _K,), jnp.int32),
            pltpu.VMEM((npn * (_K - 1),), jnp.int32),
            pltpu.VMEM((npn * (_K - 1),), jnp.int32),
            pltpu.VMEM((npn,), jnp.int32),
        ],
    )
    def sc_compact(idx_hbm, esrc_hbm, edst_hbm, deg_hbm,
                   idx_v, esrc_v, edst_v, deg_v):
        wid = lax.axis_index("s") * 2 + lax.axis_index("c")
        base = wid * npn
        pltpu.sync_copy(idx_hbm.at[pl.ds(base * _K, npn * _K)], idx_v)

        lane = lax.broadcasted_iota(jnp.int32, (_K,), 0)

        def group_body(g, carry):
            nloc = g * _K + lane
            nglob = base + nloc
            p = jnp.zeros((_K,), jnp.int32)
            c = jnp.zeros((_K,), jnp.int32)
            for j in range(_K):
                v = plsc.load_gather(idx_v, [nloc * _K + j])
                hit = v == nglob
                p = p + jnp.where(hit, j, 0)
                c = c + jnp.where(hit, 1, 0)
            p = jnp.where(c == 0, _K, p)
            deg_v[pl.ds(g * _K, _K)] = _K - c
            for t in range(_K - 1):
                jsel = t + jnp.where(p <= t, 1, 0)
                v = plsc.load_gather(idx_v, [nloc * _K + jsel])
                epos = nloc * (_K - 1) + t
                plsc.store_scatter(esrc_v, [epos], v)
                plsc.store_scatter(edst_v, [epos], nglob)
            return carry

        lax.fori_loop(0, ngr, group_body, 0)
        pltpu.sync_copy(esrc_v, esrc_hbm.at[pl.ds(base * (_K - 1),
                                                  npn * (_K - 1))])
        pltpu.sync_copy(edst_v, edst_hbm.at[pl.ds(base * (_K - 1),
                                                  npn * (_K - 1))])
        pltpu.sync_copy(deg_v, deg_hbm.at[pl.ds(base, npn)])

    return sc_compact


@jax.jit
def _knn_pallas(node_coord_src, batch_src):
    n = node_coord_src.shape[0]
    npad = ((n + _TC - 1) // _TC) * _TC
    nt = npad // _TR
    nb = 8

    coords8 = jnp.zeros((npad, 8), jnp.float32)
    coords8 = coords8.at[:n, :3].set(node_coord_src)
    coordsT = coords8.T
    brow = jnp.full((npad, 1), -1, jnp.int32).at[:n, 0].set(batch_src)
    bcol = jnp.full((1, npad), -2, jnp.int32).at[0, :n].set(batch_src)

    starts = jnp.searchsorted(batch_src, jnp.arange(nb, dtype=jnp.int32),
                              side="left").astype(jnp.int32)
    ends = jnp.searchsorted(batch_src, jnp.arange(nb, dtype=jnp.int32),
                            side="right").astype(jnp.int32)
    first = jnp.minimum(jnp.arange(nt, dtype=jnp.int32) * _TR, n - 1)
    last = jnp.minimum(first + _TR - 1, n - 1)
    lo_t = starts[batch_src[first]] // _TC
    hi_t = (ends[batch_src[last]] - 1) // _TC
    cnt_t = hi_t - lo_t + 1

    grid_spec = pltpu.PrefetchScalarGridSpec(
        num_scalar_prefetch=2,
        grid=(nt,),
        in_specs=[
            pl.BlockSpec((npad, 8), lambda i, lo, cnt: (0, 0)),
            pl.BlockSpec((8, npad), lambda i, lo, cnt: (0, 0)),
            pl.BlockSpec((npad, 1), lambda i, lo, cnt: (0, 0)),
            pl.BlockSpec((1, npad), lambda i, lo, cnt: (0, 0)),
        ],
        out_specs=[
            pl.BlockSpec((_K, _TR), lambda i, lo, cnt: (0, i)),
        ],
    )
    (out_idx,) = pl.pallas_call(
        _knn_body,
        grid_spec=grid_spec,
        out_shape=[
            jax.ShapeDtypeStruct((_K, nt * _TR), jnp.int32),
        ],
    )(lo_t, cnt_t, coords8, coordsT, brow, bcol)

    out_idx_nm = out_idx.T.reshape(-1)
    esrc, edst, deg = _make_sc_compact(npad)(out_idx_nm)
    return esrc, edst, deg


def kernel(node_coord_src, node_feature_src, batch_src):
    n = node_coord_src.shape[0]
    esrc, edst, deg = _knn_pallas(node_coord_src, batch_src)
    m = n * (_K - 1)
    return (node_feature_src, node_coord_src, esrc[:m], edst[:m], deg[:n],
            batch_src)


# speedup vs baseline: 1.1445x; 1.1445x over previous
import functools

import jax
import jax.numpy as jnp
from jax import lax
from jax.experimental import pallas as pl
from jax.experimental.pallas import tpu as pltpu
from jax.experimental.pallas import tpu_sc as plsc

_K = 16
_TR = 256
_TC = 256
_W = _K + _TC
_BIG = 2**30
_INF = jnp.inf
_NW = 32


def _knn_body(lo_ref, cnt_ref, coords_ref, coordsT_ref, brow_ref, bcol_ref,
              out_idx_ref):
    i = pl.program_id(0)
    dstT = coordsT_ref[:, pl.ds(i * _TR, _TR)]
    bd = bcol_ref[:, pl.ds(i * _TR, _TR)]
    sqd = jnp.sum(dstT * dstT, axis=0, keepdims=True)

    sub16 = jax.lax.broadcasted_iota(jnp.int32, (_K, _TR), 0)
    subW = jax.lax.broadcasted_iota(jnp.int32, (_W, _TR), 0)

    def col_body(j, carry):
        bval, bidx = carry
        c0 = (lo_ref[i] + j) * _TC
        src = coords_ref[pl.ds(c0, _TC), :]
        bs = brow_ref[pl.ds(c0, _TC), :]
        m2 = jnp.dot(src, dstT, preferred_element_type=jnp.float32)
        sqs = jnp.sum(src * src, axis=1, keepdims=True)
        d2 = (sqd + sqs) - 2.0 * m2
        d2 = jnp.where(bs == bd, d2, _INF)
        cv = jnp.concatenate([bval, d2], axis=0)
        vals, idxs = [], []
        for r in range(_K):
            m = jnp.min(cv, axis=0, keepdims=True)
            pos = jnp.argmin(cv, axis=0)[None, :]
            gi = jnp.min(jnp.where(sub16 == pos, bidx, _BIG),
                         axis=0, keepdims=True)
            vals.append(m)
            idxs.append(jnp.where(pos < _K, gi, c0 + pos - _K))
            if r < _K - 1:
                cv = jnp.where(subW == pos, _INF, cv)
        return (jnp.concatenate(vals, axis=0), jnp.concatenate(idxs, axis=0))

    init = (jnp.full((_K, _TR), _INF, jnp.float32),
            jnp.full((_K, _TR), _BIG, jnp.int32))
    _, bi = jax.lax.fori_loop(0, cnt_ref[i], col_body, init)
    out_idx_ref[...] = bi.T


def _make_sc_compact(npad):
    npn = npad // _NW
    ngr = npn // _K
    mesh = plsc.VectorSubcoreMesh(core_axis_name="c", subcore_axis_name="s")

    @functools.partial(
        pl.kernel, mesh=mesh,
        compiler_params=pltpu.CompilerParams(needs_layout_passes=False),
        out_type=[
            jax.ShapeDtypeStruct((npad * (_K - 1),), jnp.int32),
            jax.ShapeDtypeStruct((npad * (_K - 1),), jnp.int32),
            jax.ShapeDtypeStruct((npad,), jnp.int32),
        ],
        scratch_types=[
            pltpu.VMEM((npn * _K,), jnp.int32),
            pltpu.VMEM((npn * (_K - 1),), jnp.int32),
            pltpu.VMEM((npn * (_K - 1),), jnp.int32),
            pltpu.VMEM((npn,), jnp.int32),
        ],
    )
    def sc_compact(idx_hbm, esrc_hbm, edst_hbm, deg_hbm,
                   idx_v, esrc_v, edst_v, deg_v):
        wid = lax.axis_index("s") * 2 + lax.axis_index("c")
        base = wid * npn
        pltpu.sync_copy(idx_hbm.at[pl.ds(base * _K, npn * _K)], idx_v)

        lane = lax.broadcasted_iota(jnp.int32, (_K,), 0)

        def group_body(g, carry):
            nloc = g * _K + lane
            nglob = base + nloc
            p = jnp.zeros((_K,), jnp.int32)
            c = jnp.zeros((_K,), jnp.int32)
            for j in range(_K):
                v = plsc.load_gather(idx_v, [nloc * _K + j])
                hit = v == nglob
                p = p + jnp.where(hit, j, 0)
                c = c + jnp.where(hit, 1, 0)
            p = jnp.where(c == 0, _K, p)
            deg_v[pl.ds(g * _K, _K)] = _K - c
            for t in range(_K - 1):
                jsel = t + jnp.where(p <= t, 1, 0)
                v = plsc.load_gather(idx_v, [nloc * _K + jsel])
                epos = nloc * (_K - 1) + t
                plsc.store_scatter(esrc_v, [epos], v)
                plsc.store_scatter(edst_v, [epos], nglob)
            return carry

        lax.fori_loop(0, ngr, group_body, 0)
        pltpu.sync_copy(esrc_v, esrc_hbm.at[pl.ds(base * (_K - 1),
                                                  npn * (_K - 1))])
        pltpu.sync_copy(edst_v, edst_hbm.at[pl.ds(base * (_K - 1),
                                                  npn * (_K - 1))])
        pltpu.sync_copy(deg_v, deg_hbm.at[pl.ds(base, npn)])

    return sc_compact


@jax.jit
def _knn_pallas(node_coord_src, batch_src):
    n = node_coord_src.shape[0]
    npad = ((n + _TC - 1) // _TC) * _TC
    nt = npad // _TR
    nb = 8

    coords8 = jnp.zeros((npad, 8), jnp.float32)
    coords8 = coords8.at[:n, :3].set(node_coord_src)
    coordsT = coords8.T
    brow = jnp.full((npad, 1), -1, jnp.int32).at[:n, 0].set(batch_src)
    bcol = jnp.full((1, npad), -2, jnp.int32).at[0, :n].set(batch_src)

    starts = jnp.searchsorted(batch_src, jnp.arange(nb, dtype=jnp.int32),
                              side="left").astype(jnp.int32)
    ends = jnp.searchsorted(batch_src, jnp.arange(nb, dtype=jnp.int32),
                            side="right").astype(jnp.int32)
    first = jnp.minimum(jnp.arange(nt, dtype=jnp.int32) * _TR, n - 1)
    last = jnp.minimum(first + _TR - 1, n - 1)
    lo_t = starts[batch_src[first]] // _TC
    hi_t = (ends[batch_src[last]] - 1) // _TC
    cnt_t = hi_t - lo_t + 1

    grid_spec = pltpu.PrefetchScalarGridSpec(
        num_scalar_prefetch=2,
        grid=(nt,),
        in_specs=[
            pl.BlockSpec((npad, 8), lambda i, lo, cnt: (0, 0)),
            pl.BlockSpec((8, npad), lambda i, lo, cnt: (0, 0)),
            pl.BlockSpec((npad, 1), lambda i, lo, cnt: (0, 0)),
            pl.BlockSpec((1, npad), lambda i, lo, cnt: (0, 0)),
        ],
        out_specs=[
            pl.BlockSpec((_TR, _K), lambda i, lo, cnt: (i, 0)),
        ],
    )
    (out_idx,) = pl.pallas_call(
        _knn_body,
        grid_spec=grid_spec,
        out_shape=[
            jax.ShapeDtypeStruct((nt * _TR, _K), jnp.int32),
        ],
    )(lo_t, cnt_t, coords8, coordsT, brow, bcol)

    out_idx_nm = out_idx.reshape(-1)
    esrc, edst, deg = _make_sc_compact(npad)(out_idx_nm)
    return esrc, edst, deg


def kernel(node_coord_src, node_feature_src, batch_src):
    n = node_coord_src.shape[0]
    esrc, edst, deg = _knn_pallas(node_coord_src, batch_src)
    m = n * (_K - 1)
    return (node_feature_src, node_coord_src, esrc[:m], edst[:m], deg[:n],
            batch_src)


# speedup vs baseline: 1.1527x; 1.0071x over previous
import functools

import jax
import jax.numpy as jnp
from jax import lax
from jax.experimental import pallas as pl
from jax.experimental.pallas import tpu as pltpu
from jax.experimental.pallas import tpu_sc as plsc

_K = 16
_TR = 256
_TC = 256
_W = _K + _TC
_BIG = 2**30
_INF = jnp.inf
_NW = 32


def _knn_body(lo_ref, cnt_ref, coords_ref, coordsT_ref, brow_ref, bcol_ref,
              out_idx_ref):
    i = pl.program_id(0)
    dstT = coordsT_ref[:, pl.ds(i * _TR, _TR)]
    bd = bcol_ref[:, pl.ds(i * _TR, _TR)]
    sqd = jnp.sum(dstT * dstT, axis=0, keepdims=True)

    sub16 = jax.lax.broadcasted_iota(jnp.int32, (_K, _TR), 0)
    subW = jax.lax.broadcasted_iota(jnp.int32, (_W, _TR), 0)

    def col_body(j, carry):
        bval, bidx = carry
        c0 = (lo_ref[i] + j) * _TC
        src = coords_ref[pl.ds(c0, _TC), :]
        bs = brow_ref[pl.ds(c0, _TC), :]
        m2 = jnp.dot(src, dstT, preferred_element_type=jnp.float32)
        sqs = jnp.sum(src * src, axis=1, keepdims=True)
        d2 = (sqd + sqs) - 2.0 * m2
        d2 = jnp.where(bs == bd, d2, _INF)
        cv = jnp.concatenate([bval, d2], axis=0)
        vals, idxs = [], []
        for r in range(_K):
            m = jnp.min(cv, axis=0, keepdims=True)
            pos = jnp.argmin(cv, axis=0)[None, :]
            gi = jnp.min(jnp.where(sub16 == pos, bidx, _BIG),
                         axis=0, keepdims=True)
            vals.append(m)
            idxs.append(jnp.where(pos < _K, gi, c0 + pos - _K))
            if r < _K - 1:
                cv = jnp.where(subW == pos, _INF, cv)
        return (jnp.concatenate(vals, axis=0), jnp.concatenate(idxs, axis=0))

    init = (jnp.full((_K, _TR), _INF, jnp.float32),
            jnp.full((_K, _TR), _BIG, jnp.int32))
    _, bi = jax.lax.fori_loop(0, cnt_ref[i], col_body, init)
    out_idx_ref[...] = bi


def _make_sc_compact(npad):
    npn = npad // _NW
    ngr = npn // _K
    mesh = plsc.VectorSubcoreMesh(core_axis_name="c", subcore_axis_name="s")

    @functools.partial(
        pl.kernel, mesh=mesh,
        compiler_params=pltpu.CompilerParams(needs_layout_passes=False),
        out_type=[
            jax.ShapeDtypeStruct((npad * (_K - 1),), jnp.int32),
            jax.ShapeDtypeStruct((npad * (_K - 1),), jnp.int32),
            jax.ShapeDtypeStruct((npad,), jnp.int32),
        ],
        scratch_types=[
            pltpu.VMEM((npn * _K,), jnp.int32),
            pltpu.VMEM((npn * (_K - 1),), jnp.int32),
            pltpu.VMEM((npn * (_K - 1),), jnp.int32),
            pltpu.VMEM((npn,), jnp.int32),
        ],
    )
    def sc_compact(idx_hbm, esrc_hbm, edst_hbm, deg_hbm,
                   idx_v, esrc_v, edst_v, deg_v):
        wid = lax.axis_index("s") * 2 + lax.axis_index("c")
        base = wid * npn
        pltpu.sync_copy(idx_hbm.at[pl.ds(base * _K, npn * _K)], idx_v)

        lane = lax.broadcasted_iota(jnp.int32, (_K,), 0)

        def group_body(g, carry):
            nloc = g * _K + lane
            nglob = base + nloc
            p = jnp.zeros((_K,), jnp.int32)
            c = jnp.zeros((_K,), jnp.int32)
            for j in range(_K):
                v = plsc.load_gather(idx_v, [nloc * _K + j])
                hit = v == nglob
                p = p + jnp.where(hit, j, 0)
                c = c + jnp.where(hit, 1, 0)
            p = jnp.where(c == 0, _K, p)
            deg_v[pl.ds(g * _K, _K)] = _K - c
            for t in range(_K - 1):
                jsel = t + jnp.where(p <= t, 1, 0)
                v = plsc.load_gather(idx_v, [nloc * _K + jsel])
                epos = nloc * (_K - 1) + t
                plsc.store_scatter(esrc_v, [epos], v)
                plsc.store_scatter(edst_v, [epos], nglob)
            return carry

        lax.fori_loop(0, ngr, group_body, 0)
        pltpu.sync_copy(esrc_v, esrc_hbm.at[pl.ds(base * (_K - 1),
                                                  npn * (_K - 1))])
        pltpu.sync_copy(edst_v, edst_hbm.at[pl.ds(base * (_K - 1),
                                                  npn * (_K - 1))])
        pltpu.sync_copy(deg_v, deg_hbm.at[pl.ds(base, npn)])

    return sc_compact


@jax.jit
def _knn_pallas(node_coord_src, batch_src):
    n = node_coord_src.shape[0]
    npad = ((n + _TC - 1) // _TC) * _TC
    nt = npad // _TR
    nb = 8

    coords8 = jnp.zeros((npad, 8), jnp.float32)
    coords8 = coords8.at[:n, :3].set(node_coord_src)
    coordsT = coords8.T
    brow = jnp.full((npad, 1), -1, jnp.int32).at[:n, 0].set(batch_src)
    bcol = jnp.full((1, npad), -2, jnp.int32).at[0, :n].set(batch_src)

    starts = jnp.searchsorted(batch_src, jnp.arange(nb, dtype=jnp.int32),
                              side="left").astype(jnp.int32)
    ends = jnp.searchsorted(batch_src, jnp.arange(nb, dtype=jnp.int32),
                            side="right").astype(jnp.int32)
    first = jnp.minimum(jnp.arange(nt, dtype=jnp.int32) * _TR, n - 1)
    last = jnp.minimum(first + _TR - 1, n - 1)
    lo_t = starts[batch_src[first]] // _TC
    hi_t = (ends[batch_src[last]] - 1) // _TC
    cnt_t = hi_t - lo_t + 1

    grid_spec = pltpu.PrefetchScalarGridSpec(
        num_scalar_prefetch=2,
        grid=(nt,),
        in_specs=[
            pl.BlockSpec((npad, 8), lambda i, lo, cnt: (0, 0)),
            pl.BlockSpec((8, npad), lambda i, lo, cnt: (0, 0)),
            pl.BlockSpec((npad, 1), lambda i, lo, cnt: (0, 0)),
            pl.BlockSpec((1, npad), lambda i, lo, cnt: (0, 0)),
        ],
        out_specs=[
            pl.BlockSpec((_K, _TR), lambda i, lo, cnt: (0, i)),
        ],
    )
    (out_idx,) = pl.pallas_call(
        _knn_body,
        grid_spec=grid_spec,
        out_shape=[
            jax.ShapeDtypeStruct((_K, nt * _TR), jnp.int32),
        ],
    )(lo_t, cnt_t, coords8, coordsT, brow, bcol)

    out_idx_nm = out_idx.T.reshape(-1)
    esrc, edst, deg = _make_sc_compact(npad)(out_idx_nm)
    return esrc, edst, deg


def kernel(node_coord_src, node_feature_src, batch_src):
    n = node_coord_src.shape[0]
    esrc, edst, deg = _knn_pallas(node_coord_src, batch_src)
    m = n * (_K - 1)
    return (node_feature_src, node_coord_src, esrc[:m], edst[:m], deg[:n],
            batch_src)
